# baseline (speedup 1.0000x reference)
import functools

import jax
import jax.numpy as jnp
from jax import lax
from jax.experimental import pallas as pl
from jax.experimental.pallas import tpu as pltpu
from jax.experimental.pallas import tpu_sc as plsc

_B = 16384
_D = 64
_G = 8


@functools.cache
def _build_gather(num_cores: int, num_subcores: int):
    nw = num_cores * num_subcores
    b_per_w = _B // nw
    mesh = plsc.VectorSubcoreMesh(core_axis_name="c", subcore_axis_name="s")

    @functools.partial(
        pl.kernel,
        mesh=mesh,
        out_type=jax.ShapeDtypeStruct((_B, 2 * _D), jnp.float32),
        scratch_types=[
            pltpu.VMEM((b_per_w,), jnp.int32),
            pltpu.VMEM((b_per_w, 2 * _D), jnp.float32),
            pltpu.SemaphoreType.DMA,
        ],
    )
    def gather_kernel(table_hbm, idxt_hbm, out_hbm, idx_v, rows_v, sem):
        wid = lax.axis_index("s") * num_cores + lax.axis_index("c")
        base = wid * b_per_w
        pltpu.sync_copy(idxt_hbm.at[0, pl.ds(base, b_per_w)], idx_v)
        pltpu.async_copy(table_hbm.at[idx_v], rows_v, sem).wait()
        pltpu.sync_copy(rows_v, out_hbm.at[pl.ds(base, b_per_w)])

    return gather_kernel


def kernel(W, indices):
    idxt = indices.astype(jnp.int32).T
    k = 49920
    Wp = jnp.concatenate(
        [jnp.pad(W[:k], ((0, 0), (0, _D))),
         jnp.pad(W[k:], ((0, 0), (0, _D)))], axis=0)
    info = plsc.get_sparse_core_info()
    gather = _build_gather(info.num_cores, info.num_subcores)
    out_p = gather(Wp, idxt)
    return out_p[:, :_D]


# speedup vs baseline: 1.2974x; 1.2974x over previous
import functools

import jax
import jax.numpy as jnp
from jax import lax
from jax.experimental import pallas as pl
from jax.experimental.pallas import tpu as pltpu
from jax.experimental.pallas import tpu_sc as plsc

_B = 16384
_D = 64
_G = 8


@functools.cache
def _build_gather(num_cores: int, num_subcores: int):
    nw = num_cores * num_subcores
    b_per_w = _B // nw
    mesh = plsc.VectorSubcoreMesh(core_axis_name="c", subcore_axis_name="s")

    @functools.partial(
        pl.kernel,
        mesh=mesh,
        out_type=jax.ShapeDtypeStruct((_B, 2 * _D), jnp.float32),
        scratch_types=[
            pltpu.VMEM((b_per_w,), jnp.int32),
            pltpu.VMEM((b_per_w, 2 * _D), jnp.float32),
            pltpu.SemaphoreType.DMA,
        ],
    )
    def gather_kernel(table_hbm, idxt_hbm, out_hbm, idx_v, rows_v, sem):
        wid = lax.axis_index("s") * num_cores + lax.axis_index("c")
        base = wid * b_per_w
        pltpu.sync_copy(idxt_hbm.at[0, pl.ds(base, b_per_w)], idx_v)
        pltpu.async_copy(table_hbm.at[idx_v], rows_v, sem).wait()
        pltpu.sync_copy(rows_v, out_hbm.at[pl.ds(base, b_per_w)])

    return gather_kernel


def kernel(W, indices):
    idxt = indices.astype(jnp.int32).T
    Wp = jnp.pad(W, ((0, 0), (0, _D)))
    info = plsc.get_sparse_core_info()
    gather = _build_gather(info.num_cores, info.num_subcores)
    out_p = gather(Wp, idxt)
    return out_p[:, :_D]


# speedup vs baseline: 1.2986x; 1.0009x over previous
import functools

import jax
import jax.numpy as jnp
from jax import lax
from jax.experimental import pallas as pl
from jax.experimental.pallas import tpu as pltpu
from jax.experimental.pallas import tpu_sc as plsc

_B = 16384
_D = 64
_G = 8


@functools.cache
def _build_gather(num_cores: int, num_subcores: int):
    nw = num_cores * num_subcores
    b_per_w = _B // nw
    mesh = plsc.VectorSubcoreMesh(core_axis_name="c", subcore_axis_name="s")

    @functools.partial(
        pl.kernel,
        mesh=mesh,
        out_type=jax.ShapeDtypeStruct((_B, 2 * _D), jnp.float32),
        scratch_types=[
            pltpu.VMEM((b_per_w,), jnp.int32),
            pltpu.VMEM((b_per_w // 2, 2 * _D), jnp.float32),
            pltpu.VMEM((b_per_w // 2, 2 * _D), jnp.float32),
            pltpu.SemaphoreType.DMA,
            pltpu.SemaphoreType.DMA,
        ],
    )
    def gather_kernel(table_hbm, idxt_hbm, out_hbm, idx_v, rows_a, rows_b, s0, s1):
        wid = lax.axis_index("s") * num_cores + lax.axis_index("c")
        base = wid * b_per_w
        half = b_per_w // 2
        pltpu.sync_copy(idxt_hbm.at[0, pl.ds(base, b_per_w)], idx_v)
        g0 = pltpu.async_copy(table_hbm.at[idx_v.at[pl.ds(0, half)]], rows_a, s0)
        g1 = pltpu.async_copy(table_hbm.at[idx_v.at[pl.ds(half, half)]], rows_b, s1)
        g0.wait()
        pltpu.sync_copy(rows_a, out_hbm.at[pl.ds(base, half)])
        g1.wait()
        pltpu.sync_copy(rows_b, out_hbm.at[pl.ds(base + half, half)])

    return gather_kernel


def kernel(W, indices):
    idxt = indices.astype(jnp.int32).T
    Wp = jnp.pad(W, ((0, 0), (0, _D)))
    info = plsc.get_sparse_core_info()
    gather = _build_gather(info.num_cores, info.num_subcores)
    out_p = gather(Wp, idxt)
    return out_p[:, :_D]


# speedup vs baseline: 1.3007x; 1.0016x over previous
import functools

import jax
import jax.numpy as jnp
from jax import lax
from jax.experimental import pallas as pl
from jax.experimental.pallas import tpu as pltpu
from jax.experimental.pallas import tpu_sc as plsc

_B = 16384
_D = 64


@functools.cache
def _build_gather(num_cores: int, num_subcores: int):
    nw = num_cores * num_subcores
    b_per_w = _B // nw
    mesh = plsc.VectorSubcoreMesh(core_axis_name="c", subcore_axis_name="s")

    @functools.partial(
        pl.kernel,
        mesh=mesh,
        out_type=jax.ShapeDtypeStruct((_B, 2 * _D), jnp.float32),
        scratch_types=[
            pltpu.VMEM((b_per_w,), jnp.int32),
            pltpu.VMEM((b_per_w // 2, 2 * _D), jnp.float32),
            pltpu.VMEM((b_per_w // 2, 2 * _D), jnp.float32),
            pltpu.SemaphoreType.DMA,
            pltpu.SemaphoreType.DMA,
        ],
    )
    def gather_kernel(table_hbm, idxt_hbm, out_hbm, idx_v, rows_a, rows_b, s0, s1):
        wid = lax.axis_index("s") * num_cores + lax.axis_index("c")
        base = wid * b_per_w
        half = b_per_w // 2
        pltpu.sync_copy(idxt_hbm.at[0, pl.ds(base, b_per_w)], idx_v)
        g0 = pltpu.async_copy(table_hbm.at[idx_v.at[pl.ds(0, half)]], rows_a, s0)
        g1 = pltpu.async_copy(table_hbm.at[idx_v.at[pl.ds(half, half)]], rows_b, s1)
        g0.wait()
        pltpu.sync_copy(rows_a, out_hbm.at[pl.ds(base, half)])
        g1.wait()
        pltpu.sync_copy(rows_b, out_hbm.at[pl.ds(base + half, half)])

    return gather_kernel


def kernel(W, indices):
    idxt = indices.astype(jnp.int32).T
    Wp = jnp.pad(W, ((0, 0), (0, _D)))
    info = plsc.get_sparse_core_info()
    gather = _build_gather(info.num_cores, info.num_subcores)
    out_p = gather(Wp, idxt)
    return out_p[:, :_D]
